# baseline (speedup 1.0000x reference)
import math

import jax
import jax.numpy as jnp
from jax.experimental import pallas as pl
from jax.experimental.pallas import tpu as pltpu

G = 512
A = 32
N = G * A
H = 128
F = 64
R = 12
CUT = 5.0
NAF = 13
NMF = 8
OUT = 256
NTYPES = 101
EMB = 5

GB = 8
M = GB * A
P = M * A

_INTERPRET = False


def _block_kernel(x_ref, pos_ref, T_ref, Wn_ref, bn_ref,
                  Wh0_ref, Wr0_ref, Wuh0_ref, Wua0_ref, bu0_ref,
                  Wh1_ref, Wr1_ref, Wuh1_ref, Wua1_ref, bu1_ref,
                  Wh2_ref, Wr2_ref, Wuh2_ref, Wua2_ref, bu2_ref,
                  Wmol_ref, bmol_ref, W1g_ref, W1m_ref, bf1_ref,
                  W2_ref, bf2_ref, Wo_ref, out_ref):
    gelu = jax.nn.gelu
    f32 = jnp.float32

    xb = x_ref[...]
    posb = pos_ref[...]

    row = jax.lax.broadcasted_iota(jnp.int32, (M, 1), 0)
    first = (row % A == 0).astype(f32)
    molx = jnp.sum((xb * first).reshape(GB, A, NAF), axis=1)
    mol = jnp.dot(molx[:, NAF - NMF:], Wmol_ref[...],
                  preferred_element_type=f32) + bmol_ref[...]

    atype = jnp.clip((xb[:, 0:1] * NTYPES).astype(jnp.int32), 0, NTYPES - 1)
    lanes = jax.lax.broadcasted_iota(jnp.int32, (M, 128), 1)
    onehot = (lanes == atype).astype(f32)
    h = gelu(jnp.dot(onehot, T_ref[...], preferred_element_type=f32)
             + jnp.dot(xb[:, 1:], Wn_ref[...], preferred_element_type=f32)
             + bn_ref[...])

    prow = jnp.broadcast_to(posb.reshape(M, 1, 3), (M, A, 3)).reshape(P, 3)
    pcol = jnp.broadcast_to(posb.reshape(GB, 1, A, 3), (GB, A, A, 3)).reshape(P, 3)
    df = prow - pcol
    d = jnp.sqrt(jnp.sum(df * df, axis=1, keepdims=True) + 1e-12)
    ridx = jax.lax.broadcasted_iota(jnp.int32, (P, 1), 0)
    jj = ridx % A
    ii = (ridx // A) % A
    adj = (d < CUT) & (ii != jj)
    adjf = adj.astype(f32)
    dsafe = jnp.where(adj, d, 1.0)
    kf = (jax.lax.broadcasted_iota(jnp.int32, (1, R), 1) + 1).astype(f32)
    rbf = (math.sqrt(2.0 / CUT) * jnp.sin(kf * math.pi * dsafe / CUT)
           / dsafe) * adjf

    for (Wh_ref, Wr_ref, Wuh_ref, Wua_ref, bu_ref) in (
            (Wh0_ref, Wr0_ref, Wuh0_ref, Wua0_ref, bu0_ref),
            (Wh1_ref, Wr1_ref, Wuh1_ref, Wua1_ref, bu1_ref),
            (Wh2_ref, Wr2_ref, Wuh2_ref, Wua2_ref, bu2_ref)):
        q = jnp.dot(h, Wh_ref[...], preferred_element_type=f32)
        qt = jnp.broadcast_to(q.reshape(GB, 1, A, F),
                              (GB, A, A, F)).reshape(P, F)
        pre = qt + jnp.dot(rbf, Wr_ref[...], preferred_element_type=f32)
        m = gelu(pre) * adjf
        agg = jnp.sum(m.reshape(M, A, F), axis=1)
        upd = gelu(jnp.dot(h, Wuh_ref[...], preferred_element_type=f32)
                   + jnp.dot(agg, Wua_ref[...], preferred_element_type=f32)
                   + bu_ref[...])
        h = h + upd

    xg = jnp.sum(h.reshape(GB, A, H), axis=1) * (1.0 / A)
    z = gelu(jnp.dot(xg, W1g_ref[...], preferred_element_type=f32)
             + jnp.dot(mol, W1m_ref[...], preferred_element_type=f32)
             + bf1_ref[...])
    z = gelu(jnp.dot(z, W2_ref[...], preferred_element_type=f32) + bf2_ref[...])
    out_ref[...] = jnp.dot(z, Wo_ref[...], preferred_element_type=f32)


def kernel(x, pos, batch, ptr, aux_ind, num_graphs, atom_emb, W_node, b_node,
           Wh0, Wr0, Wu0, bu0, Wh1, Wr1, Wu1, bu1, Wh2, Wr2, Wu2, bu2,
           W_mol, b_mol, W_fc1, b_fc1, W_fc2, b_fc2, W_out):
    f32 = jnp.float32
    T = jnp.zeros((128, H), f32).at[:NTYPES].set(
        atom_emb @ W_node[:EMB])
    Wn = W_node[EMB:]

    row_specs = [
        pl.BlockSpec((M, NAF), lambda g: (g, 0)),
        pl.BlockSpec((M, 3), lambda g: (g, 0)),
    ]

    full = lambda a: pl.BlockSpec(a.shape, lambda g: tuple(0 for _ in a.shape))
    weights = [T, Wn, b_node.reshape(1, H),
               Wh0, Wr0, Wu0[:H], Wu0[H:], bu0.reshape(1, H),
               Wh1, Wr1, Wu1[:H], Wu1[H:], bu1.reshape(1, H),
               Wh2, Wr2, Wu2[:H], Wu2[H:], bu2.reshape(1, H),
               W_mol, b_mol.reshape(1, NMF),
               W_fc1[:H], W_fc1[H:], b_fc1.reshape(1, H),
               W_fc2, b_fc2.reshape(1, H), W_out]

    out = pl.pallas_call(
        _block_kernel,
        grid=(G // GB,),
        in_specs=row_specs + [full(w) for w in weights],
        out_specs=pl.BlockSpec((GB, OUT), lambda g: (g, 0)),
        out_shape=jax.ShapeDtypeStruct((G, OUT), f32),
        compiler_params=pltpu.CompilerParams(
            dimension_semantics=("arbitrary",)),
        interpret=_INTERPRET,
    )(x, pos, *weights)
    return out


# speedup vs baseline: 1.8407x; 1.8407x over previous
import math

import jax
import jax.numpy as jnp
from jax.experimental import pallas as pl
from jax.experimental.pallas import tpu as pltpu

G = 512
A = 32
N = G * A
H = 128
F = 64
R = 12
CUT = 5.0
NAF = 13
NMF = 8
OUT = 256
NTYPES = 101
EMB = 5

GB = 8
M = GB * A
P = M * A

_INTERPRET = False


def _block_kernel(x_ref, pos_ref, T_ref, Wn_ref, bn_ref,
                  Wh0_ref, Wr0_ref, Wuh0_ref, Wua0_ref, bu0_ref,
                  Wh1_ref, Wr1_ref, Wuh1_ref, Wua1_ref, bu1_ref,
                  Wh2_ref, Wr2_ref, Wuh2_ref, Wua2_ref, bu2_ref,
                  Wmol_ref, bmol_ref, W1g_ref, W1m_ref, bf1_ref,
                  W2_ref, bf2_ref, Wo_ref, out_ref):
    gelu = jax.nn.gelu
    f32 = jnp.float32

    xb = x_ref[...]
    posb = pos_ref[...]

    row = jax.lax.broadcasted_iota(jnp.int32, (M, 1), 0)
    first = (row % A == 0).astype(f32)
    molx = jnp.sum((xb * first).reshape(GB, A, NAF), axis=1)
    mol = jnp.dot(molx[:, NAF - NMF:], Wmol_ref[...],
                  preferred_element_type=f32) + bmol_ref[...]

    atype = jnp.clip((xb[:, 0:1] * NTYPES).astype(jnp.int32), 0, NTYPES - 1)
    lanes = jax.lax.broadcasted_iota(jnp.int32, (M, 128), 1)
    onehot = (lanes == atype).astype(f32)
    h = gelu(jnp.dot(onehot, T_ref[...], preferred_element_type=f32)
             + jnp.dot(xb[:, 1:], Wn_ref[...], preferred_element_type=f32)
             + bn_ref[...])

    prow = jnp.broadcast_to(posb.reshape(M, 1, 3), (M, A, 3)).reshape(P, 3)
    pcol = jnp.broadcast_to(posb.reshape(GB, 1, A, 3), (GB, A, A, 3)).reshape(P, 3)
    df = prow - pcol
    d = jnp.sqrt(jnp.sum(df * df, axis=1, keepdims=True) + 1e-12)
    ridx = jax.lax.broadcasted_iota(jnp.int32, (P, 1), 0)
    jj = ridx % A
    ii = (ridx // A) % A
    adj = (d < CUT) & (ii != jj)
    adjf = adj.astype(f32)
    dsafe = jnp.where(adj, d, 1.0)
    kf = (jax.lax.broadcasted_iota(jnp.int32, (1, R), 1) + 1).astype(f32)
    theta = dsafe * (kf * (math.pi / CUT))
    n = jnp.round(theta * (0.5 / math.pi))
    v = theta - n * (2.0 * math.pi)
    v2 = v * v
    s = v * (0.9999994441442891 + v2 * (-0.1666651950620369 + v2 * (
        0.00833220729172304 + v2 * (-0.00019803942981621122 + v2 * (
            2.694818791282763e-06 + v2 * -2.0177080094133367e-08)))))
    amp = jnp.where(adj, math.sqrt(2.0 / CUT) / d, 0.0)
    rbf = s * amp

    for (Wh_ref, Wr_ref, Wuh_ref, Wua_ref, bu_ref) in (
            (Wh0_ref, Wr0_ref, Wuh0_ref, Wua0_ref, bu0_ref),
            (Wh1_ref, Wr1_ref, Wuh1_ref, Wua1_ref, bu1_ref),
            (Wh2_ref, Wr2_ref, Wuh2_ref, Wua2_ref, bu2_ref)):
        q = jnp.dot(h, Wh_ref[...], preferred_element_type=f32)
        qt = jnp.broadcast_to(q.reshape(GB, 1, A, F),
                              (GB, A, A, F)).reshape(P, F)
        pre = qt + jnp.dot(rbf, Wr_ref[...], preferred_element_type=f32)
        m = gelu(pre) * adjf
        agg = jnp.sum(m.reshape(M, A, F), axis=1)
        upd = gelu(jnp.dot(h, Wuh_ref[...], preferred_element_type=f32)
                   + jnp.dot(agg, Wua_ref[...], preferred_element_type=f32)
                   + bu_ref[...])
        h = h + upd

    xg = jnp.sum(h.reshape(GB, A, H), axis=1) * (1.0 / A)
    z = gelu(jnp.dot(xg, W1g_ref[...], preferred_element_type=f32)
             + jnp.dot(mol, W1m_ref[...], preferred_element_type=f32)
             + bf1_ref[...])
    z = gelu(jnp.dot(z, W2_ref[...], preferred_element_type=f32) + bf2_ref[...])
    out_ref[...] = jnp.dot(z, Wo_ref[...], preferred_element_type=f32)


def kernel(x, pos, batch, ptr, aux_ind, num_graphs, atom_emb, W_node, b_node,
           Wh0, Wr0, Wu0, bu0, Wh1, Wr1, Wu1, bu1, Wh2, Wr2, Wu2, bu2,
           W_mol, b_mol, W_fc1, b_fc1, W_fc2, b_fc2, W_out):
    f32 = jnp.float32
    T = jnp.zeros((128, H), f32).at[:NTYPES].set(
        atom_emb @ W_node[:EMB])
    Wn = W_node[EMB:]

    row_specs = [
        pl.BlockSpec((M, NAF), lambda g: (g, 0)),
        pl.BlockSpec((M, 3), lambda g: (g, 0)),
    ]

    full = lambda a: pl.BlockSpec(a.shape, lambda g: tuple(0 for _ in a.shape))
    weights = [T, Wn, b_node.reshape(1, H),
               Wh0, Wr0, Wu0[:H], Wu0[H:], bu0.reshape(1, H),
               Wh1, Wr1, Wu1[:H], Wu1[H:], bu1.reshape(1, H),
               Wh2, Wr2, Wu2[:H], Wu2[H:], bu2.reshape(1, H),
               W_mol, b_mol.reshape(1, NMF),
               W_fc1[:H], W_fc1[H:], b_fc1.reshape(1, H),
               W_fc2, b_fc2.reshape(1, H), W_out]

    out = pl.pallas_call(
        _block_kernel,
        grid=(G // GB,),
        in_specs=row_specs + [full(w) for w in weights],
        out_specs=pl.BlockSpec((GB, OUT), lambda g: (g, 0)),
        out_shape=jax.ShapeDtypeStruct((G, OUT), f32),
        compiler_params=pltpu.CompilerParams(
            dimension_semantics=("arbitrary",)),
        interpret=_INTERPRET,
    )(x, pos, *weights)
    return out


# speedup vs baseline: 2.5674x; 1.3948x over previous
import math

import jax
import jax.numpy as jnp
from jax.experimental import pallas as pl
from jax.experimental.pallas import tpu as pltpu

G = 512
A = 32
N = G * A
H = 128
F = 64
R = 12
CUT = 5.0
NAF = 13
NMF = 8
OUT = 256
NTYPES = 101
EMB = 5

GB = 8
M = GB * A
PH = M * A // 2
AH = A // 2

_INTERPRET = False

_C0 = math.sqrt(2.0 / CUT)


def _block_kernel(x_ref, pos_ref, posc_ref, T_ref, Wn_ref, bn_ref,
                  Wh0_ref, Wr0_ref, Wuh0_ref, Wua0_ref, bu0_ref,
                  Wh1_ref, Wr1_ref, Wuh1_ref, Wua1_ref, bu1_ref,
                  Wh2_ref, Wr2_ref, Wuh2_ref, Wua2_ref, bu2_ref,
                  Wmol_ref, bmol_ref, W1g_ref, W1m_ref, bf1_ref,
                  W2_ref, bf2_ref, Wo_ref, out_ref):
    gelu = jax.nn.gelu
    f32 = jnp.float32
    i32 = jnp.int32

    xb = x_ref[...]
    posb = pos_ref[...]
    poscb = posc_ref[...]

    row = jax.lax.broadcasted_iota(i32, (M, 1), 0)
    first = (row % A == 0).astype(f32)
    molx = jnp.sum((xb * first).reshape(GB, A, NAF), axis=1)
    mol = jnp.dot(molx[:, NAF - NMF:], Wmol_ref[...],
                  preferred_element_type=f32) + bmol_ref[...]

    atype = jnp.clip((xb[:, 0:1] * NTYPES).astype(i32), 0, NTYPES - 1)
    lanes = jax.lax.broadcasted_iota(i32, (M, 128), 1)
    onehot = (lanes == atype).astype(f32)
    h = gelu(jnp.dot(onehot, T_ref[...], preferred_element_type=f32)
             + jnp.dot(xb[:, 1:], Wn_ref[...], preferred_element_type=f32)
             + bn_ref[...])

    posb6 = jnp.concatenate([posb, posb], axis=1)
    prow = jnp.broadcast_to(posb6.reshape(M, 1, 6),
                            (M, AH, 6)).reshape(PH, 6)
    pcol = jnp.broadcast_to(poscb.reshape(GB, 1, AH, 6),
                            (GB, A, AH, 6)).reshape(PH, 6)
    df = prow - pcol
    sq = df * df
    rep0 = jax.lax.broadcasted_iota(i32, (6, 2 * R), 0)
    rep1 = jax.lax.broadcasted_iota(i32, (6, 2 * R), 1)
    REP = (rep0 // 3 == rep1 // R).astype(f32)
    d2rep = jnp.dot(sq, REP, preferred_element_type=f32)
    drep = jnp.sqrt(d2rep + 1e-12)

    ridx = jax.lax.broadcasted_iota(i32, (PH, 1), 0)
    jp = ridx % AH
    ii = (ridx // AH) % A
    d_e = drep[:, 0:1]
    d_o = drep[:, R:R + 1]
    amp_e = jnp.where((d_e < CUT) & (ii != 2 * jp), _C0 / d_e, 0.0)
    amp_o = jnp.where((d_o < CUT) & (ii != 2 * jp + 1), _C0 / d_o, 0.0)
    acol = jnp.concatenate([amp_e, amp_o], axis=1)
    r2a = jax.lax.broadcasted_iota(i32, (2, 128), 0)
    r2b = jax.lax.broadcasted_iota(i32, (2, 128), 1)
    REP2 = (r2a == r2b // F).astype(f32)
    ampR = jnp.dot(acol, REP2, preferred_element_type=f32)
    penR = jnp.where(ampR > 0.0, 0.0, -200.0)

    kf2 = ((jax.lax.broadcasted_iota(i32, (1, 2 * R), 1) % R + 1)
           .astype(f32) * (math.pi / CUT))
    theta = drep * kf2
    n = jnp.round(theta * (0.5 / math.pi))
    v = theta - n * (2.0 * math.pi)
    v2 = v * v
    s = v * (0.9999994441442891 + v2 * (-0.1666651950620369 + v2 * (
        0.00833220729172304 + v2 * (-0.00019803942981621122 + v2 * (
            2.694818791282763e-06 + v2 * -2.0177080094133367e-08)))))

    se0 = jax.lax.broadcasted_iota(i32, (M // 2, M), 0)
    se1 = jax.lax.broadcasted_iota(i32, (M // 2, M), 1)
    SELE = (2 * se0 == se1).astype(f32)
    SELO = (2 * se0 + 1 == se1).astype(f32)

    for (Wh_ref, Wr_ref, Wuh_ref, Wua_ref, bu_ref) in (
            (Wh0_ref, Wr0_ref, Wuh0_ref, Wua0_ref, bu0_ref),
            (Wh1_ref, Wr1_ref, Wuh1_ref, Wua1_ref, bu1_ref),
            (Wh2_ref, Wr2_ref, Wuh2_ref, Wua2_ref, bu2_ref)):
        q = jnp.dot(h, Wh_ref[...], preferred_element_type=f32)
        q2 = jnp.concatenate(
            [jnp.dot(SELE, q, preferred_element_type=f32),
             jnp.dot(SELO, q, preferred_element_type=f32)], axis=1)
        qt = jnp.broadcast_to(q2.reshape(GB, 1, AH, 2 * F),
                              (GB, A, AH, 2 * F)).reshape(PH, 2 * F)
        z2 = jnp.dot(s, Wr_ref[...], preferred_element_type=f32)
        m = gelu(qt + z2 * ampR + penR)
        sj = jnp.sum(m.reshape(M, AH, 2 * F), axis=1)
        agg = sj[:, :F] + sj[:, F:]
        upd = gelu(jnp.dot(h, Wuh_ref[...], preferred_element_type=f32)
                   + jnp.dot(agg, Wua_ref[...], preferred_element_type=f32)
                   + bu_ref[...])
        h = h + upd

    xg = jnp.sum(h.reshape(GB, A, H), axis=1) * (1.0 / A)
    z = gelu(jnp.dot(xg, W1g_ref[...], preferred_element_type=f32)
             + jnp.dot(mol, W1m_ref[...], preferred_element_type=f32)
             + bf1_ref[...])
    z = gelu(jnp.dot(z, W2_ref[...], preferred_element_type=f32) + bf2_ref[...])
    out_ref[...] = jnp.dot(z, Wo_ref[...], preferred_element_type=f32)


def kernel(x, pos, batch, ptr, aux_ind, num_graphs, atom_emb, W_node, b_node,
           Wh0, Wr0, Wu0, bu0, Wh1, Wr1, Wu1, bu1, Wh2, Wr2, Wu2, bu2,
           W_mol, b_mol, W_fc1, b_fc1, W_fc2, b_fc2, W_out):
    f32 = jnp.float32
    T = jnp.zeros((128, H), f32).at[:NTYPES].set(
        atom_emb @ W_node[:EMB])
    Wn = W_node[EMB:]
    posc = pos.reshape(G, A // 2, 6)

    def blockdiag2(W):
        Z = jnp.zeros((2 * R, 2 * F), f32)
        return Z.at[:R, :F].set(W).at[R:, F:].set(W)

    row_specs = [
        pl.BlockSpec((M, NAF), lambda g: (g, 0)),
        pl.BlockSpec((M, 3), lambda g: (g, 0)),
        pl.BlockSpec((GB, A // 2, 6), lambda g: (g, 0, 0)),
    ]

    full = lambda a: pl.BlockSpec(a.shape, lambda g: tuple(0 for _ in a.shape))
    weights = [T, Wn, b_node.reshape(1, H),
               Wh0, blockdiag2(Wr0), Wu0[:H], Wu0[H:], bu0.reshape(1, H),
               Wh1, blockdiag2(Wr1), Wu1[:H], Wu1[H:], bu1.reshape(1, H),
               Wh2, blockdiag2(Wr2), Wu2[:H], Wu2[H:], bu2.reshape(1, H),
               W_mol, b_mol.reshape(1, NMF),
               W_fc1[:H], W_fc1[H:], b_fc1.reshape(1, H),
               W_fc2, b_fc2.reshape(1, H), W_out]

    out = pl.pallas_call(
        _block_kernel,
        grid=(G // GB,),
        in_specs=row_specs + [full(w) for w in weights],
        out_specs=pl.BlockSpec((GB, OUT), lambda g: (g, 0)),
        out_shape=jax.ShapeDtypeStruct((G, OUT), f32),
        compiler_params=pltpu.CompilerParams(
            dimension_semantics=("arbitrary",)),
        interpret=_INTERPRET,
    )(x, pos, posc, *weights)
    return out
